# baseline (speedup 1.0000x reference)
import functools

import jax
import jax.numpy as jnp
from jax import lax
from jax.experimental import pallas as pl
from jax.experimental.pallas import tpu as pltpu
from jax.experimental.pallas import tpu_sc as plsc

_B = 256
_D = 128
_K = 4096
_T = 0.07
_INV_T = 1.0 / _T

_NW = 32
_QW = _B // _NW
_CH = 128
_NCH = _K // _CH
_LANES = 16
_DV = _D // _LANES


def _prologue_body(fq_ref, fk_ref, fqs_ref, fkn_ref, dl_ref):
    fq = fq_ref[...]
    fk = fk_ref[...]
    nq = jnp.sqrt(jnp.sum(fq * fq, axis=1, keepdims=True))
    nk = jnp.sqrt(jnp.sum(fk * fk, axis=1, keepdims=True))
    fqn = fq / jnp.maximum(nq, 1e-12)
    fkn = fk / jnp.maximum(nk, 1e-12)
    fkn_ref[...] = fkn
    fqs_ref[...] = fqn * _INV_T
    dl_ref[...] = jnp.sum(fqn * fkn, axis=1, keepdims=True) * _INV_T


def _epilogue_body(m_ref, s_ref, dl_ref, loss_ref):
    m_n = m_ref[...]
    s_n = s_ref[...]
    dl = dl_ref[...]
    m_t = jnp.maximum(m_n, dl)
    s_t = s_n * jnp.exp(m_n - m_t) + jnp.exp(dl - m_t)
    val = jnp.log(s_t) + m_t - dl
    loss_ref[...] = jnp.sum(val, axis=0, keepdims=True) * (1.0 / _B)


def _sc_body(fqs_hbm, nidx_hbm, bank_hbm, m_hbm, s_hbm,
             fq_v, idx_v, bufs, log_v, mx_v, mo_v, so_v, sems):
    wid = lax.axis_index("s") * 2 + lax.axis_index("c")
    qbase = wid * _QW
    pltpu.sync_copy(fqs_hbm.at[pl.ds(qbase, _QW)], fq_v)
    pltpu.sync_copy(nidx_hbm.at[wid], idx_v)
    lanes = lax.iota(jnp.int32, 16)
    zero16 = jnp.zeros((16,), jnp.float32)
    neg16 = jnp.full((16,), -jnp.inf, jnp.float32)

    mo_v[...] = zero16
    so_v[...] = zero16

    def _compute_chunk(qq, cc, buf):
        fqv = [fq_v[qq, pl.ds(16 * j, 16)] for j in range(_DV)]

        @pl.loop(0, _CH // 16)
        def _group(g):
            acc = zero16
            for i in range(16):
                r = g * 16 + i
                v = buf[r, pl.ds(0, 16)] * fqv[0]
                for j in range(1, _DV):
                    v = v + buf[r, pl.ds(16 * j, 16)] * fqv[j]
                acc = jnp.where(lanes == i, jnp.sum(v), acc)
            log_v[pl.ds(cc * _CH + g * 16, 16)] = acc
            mx_v[...] = jnp.maximum(mx_v[...], acc)

    def _start(t, k):
        for h in range(2):
            pltpu.async_copy(
                bank_hbm.at[pl.ds(t * _CH + 64 * h, 64)],
                bufs.at[k, pl.ds(64 * h, 64)], sems.at[2 * k + h])

    def _wait(t, k):
        for h in range(2):
            pltpu.make_async_copy(
                bank_hbm.at[pl.ds(t * _CH + 64 * h, 64)],
                bufs.at[k, pl.ds(64 * h, 64)], sems.at[2 * k + h]).wait()

    _NT = _QW * _NCH
    for k in range(3):
        _start(k, k)

    @pl.loop(0, _NT, step=4)
    def _chunk(t0):
        for k in range(4):
            t = t0 + k
            qq = t // _NCH
            cc = t % _NCH

            @pl.when(cc == 0)
            def _():
                mx_v[...] = neg16

            @pl.when(t + 3 < _NT)
            def _():
                _start(t + 3, (k + 3) % 4)

            _wait(t, k)
            _compute_chunk(qq, cc, bufs.at[k])

            @pl.when(cc == _NCH - 1)
            def _():
                m_q = jnp.max(mx_v[...])
                svec = pl.loop(0, _K // 16, init_carry=zero16)(
                    lambda j, s: s + jnp.exp(log_v[pl.ds(16 * j, 16)] - m_q))
                s_q = jnp.sum(svec)
                sel = lanes == qq
                mo_v[...] = jnp.where(sel, m_q, mo_v[...])
                so_v[...] = jnp.where(sel, s_q, so_v[...])

    pltpu.sync_copy(mo_v, m_hbm.at[wid])
    pltpu.sync_copy(so_v, s_hbm.at[wid])


_sc_kernel = functools.partial(
    pl.kernel,
    out_type=(jax.ShapeDtypeStruct((_NW, 16), jnp.float32),
              jax.ShapeDtypeStruct((_NW, 16), jnp.float32)),
    mesh=plsc.VectorSubcoreMesh(core_axis_name="c", subcore_axis_name="s"),
    compiler_params=pltpu.CompilerParams(needs_layout_passes=False),
    scratch_types=[
        pltpu.VMEM((_QW, _D), jnp.float32),
        pltpu.VMEM((_QW, _NCH, _CH), jnp.int32),
        pltpu.VMEM((4, _CH, _D), jnp.float32),
        pltpu.VMEM((_K,), jnp.float32),
        pltpu.VMEM((16,), jnp.float32),
        pltpu.VMEM((16,), jnp.float32),
        pltpu.VMEM((16,), jnp.float32),
        pltpu.SemaphoreType.DMA((8,)),
    ],
)(_sc_body)


def kernel(idx, fea_q, fea_k, gpu_idx, bank, noise_idx):
    del idx, gpu_idx
    fqs, fkn, dl = pl.pallas_call(
        _prologue_body,
        out_shape=(
            jax.ShapeDtypeStruct((_B, _D), jnp.float32),
            jax.ShapeDtypeStruct((_B, _D), jnp.float32),
            jax.ShapeDtypeStruct((_B, 1), jnp.float32),
        ),
    )(fea_q, fea_k)

    nidx3 = noise_idx.reshape(_NW, _QW, _NCH, _CH)
    m, s = _sc_kernel(fqs, nidx3, bank)
    m = m[:, :_QW].reshape(_B, 1)
    s = s[:, :_QW].reshape(_B, 1)

    loss = pl.pallas_call(
        _epilogue_body,
        out_shape=jax.ShapeDtypeStruct((1, 1), jnp.float32),
    )(m, s, dl)
    return (loss[0, 0], fkn)


# speedup vs baseline: 1.2582x; 1.2582x over previous
import functools

import jax
import jax.numpy as jnp
from jax import lax
from jax.experimental import pallas as pl
from jax.experimental.pallas import tpu as pltpu
from jax.experimental.pallas import tpu_sc as plsc

_B = 256
_D = 128
_K = 4096
_T = 0.07
_INV_T = 1.0 / _T

_NW = 32
_QW = _B // _NW
_CH = 128
_NCH = _K // _CH
_LANES = 16
_DV = _D // _LANES


def _prologue_body(fq_ref, fk_ref, fkn_ref, dl_ref):
    fq = fq_ref[...]
    fk = fk_ref[...]
    nq = jnp.sqrt(jnp.sum(fq * fq, axis=1, keepdims=True))
    nk = jnp.sqrt(jnp.sum(fk * fk, axis=1, keepdims=True))
    fqn = fq / jnp.maximum(nq, 1e-12)
    fkn = fk / jnp.maximum(nk, 1e-12)
    fkn_ref[...] = fkn
    dl_ref[...] = jnp.sum(fqn * fkn, axis=1, keepdims=True) * _INV_T


def _epilogue_body(m_ref, s_ref, dl_ref, loss_ref):
    m_n = m_ref[...]
    s_n = s_ref[...]
    dl = dl_ref[...]
    m_t = jnp.maximum(m_n, dl)
    s_t = s_n * jnp.exp(m_n - m_t) + jnp.exp(dl - m_t)
    val = jnp.log(s_t) + m_t - dl
    loss_ref[...] = jnp.sum(val, axis=0, keepdims=True) * (1.0 / _B)


def _sc_body(fqs_hbm, nidx_hbm, bank_hbm, m_hbm, s_hbm,
             fq_v, idx_v, bufs, log_v, mx_v, mo_v, so_v, sems):
    wid = lax.axis_index("s") * 2 + lax.axis_index("c")
    qbase = wid * _QW
    pltpu.sync_copy(fqs_hbm.at[pl.ds(qbase, _QW)], fq_v)
    pltpu.sync_copy(nidx_hbm.at[wid, 0], idx_v.at[0])
    lanes = lax.iota(jnp.int32, 16)
    zero16 = jnp.zeros((16,), jnp.float32)
    neg16 = jnp.full((16,), -jnp.inf, jnp.float32)

    mo_v[...] = zero16
    so_v[...] = zero16

    def _compute_chunk(qq, cc, buf):
        fqv = [fq_v[qq, pl.ds(16 * j, 16)] for j in range(_DV)]

        @pl.loop(0, _CH // 16)
        def _group(g):
            acc = zero16
            for i in range(16):
                r = g * 16 + i
                v = buf[r, pl.ds(0, 16)] * fqv[0]
                for j in range(1, _DV):
                    v = v + buf[r, pl.ds(16 * j, 16)] * fqv[j]
                acc = jnp.where(lanes == i, jnp.sum(v), acc)
            log_v[pl.ds(cc * _CH + g * 16, 16)] = acc
            mx_v[...] = jnp.maximum(mx_v[...], acc)

    def _start(t, k):
        qq, cc = t // _NCH, t % _NCH
        for h in range(2):
            pltpu.async_copy(
                bank_hbm.at[idx_v.at[qq, cc, pl.ds(64 * h, 64)]],
                bufs.at[k, pl.ds(64 * h, 64)], sems.at[2 * k + h])

    def _wait(t, k):
        qq, cc = t // _NCH, t % _NCH
        for h in range(2):
            pltpu.make_async_copy(
                bank_hbm.at[idx_v.at[qq, cc, pl.ds(64 * h, 64)]],
                bufs.at[k, pl.ds(64 * h, 64)], sems.at[2 * k + h]).wait()

    _NT = _QW * _NCH
    for k in range(3):
        _start(k, k)
    pltpu.sync_copy(nidx_hbm.at[wid, pl.ds(1, _QW - 1)],
                    idx_v.at[pl.ds(1, _QW - 1)])

    @pl.loop(0, _NT, step=4)
    def _chunk(t0):
        for k in range(4):
            t = t0 + k
            qq = t // _NCH
            cc = t % _NCH

            @pl.when(cc == 0)
            def _():
                mx_v[...] = neg16

            @pl.when(t + 3 < _NT)
            def _():
                _start(t + 3, (k + 3) % 4)

            _wait(t, k)
            _compute_chunk(qq, cc, bufs.at[k])

            @pl.when(cc == _NCH - 1)
            def _():
                fqv = [fq_v[qq, pl.ds(16 * j, 16)] for j in range(_DV)]
                nv = fqv[0] * fqv[0]
                for j in range(1, _DV):
                    nv = nv + fqv[j] * fqv[j]
                n2 = jnp.maximum(jnp.sum(nv), 1e-24)
                nvec = jnp.full((16,), n2, jnp.float32)
                iv = plsc.bitcast(nvec, jnp.int32)
                y = plsc.bitcast(
                    jnp.int32(0x5F3759DF) - (iv >> 1), jnp.float32)
                for _ in range(3):
                    y = y * (1.5 - 0.5 * nvec * y * y)
                sc = (y * _INV_T)[0]
                m_q = jnp.max(mx_v[...]) * sc
                svec = pl.loop(0, _K // 16, init_carry=zero16)(
                    lambda j, s:
                        s + jnp.exp(log_v[pl.ds(16 * j, 16)] * sc - m_q))
                s_q = jnp.sum(svec)
                sel = lanes == qq
                mo_v[...] = jnp.where(sel, m_q, mo_v[...])
                so_v[...] = jnp.where(sel, s_q, so_v[...])

    pltpu.sync_copy(mo_v, m_hbm.at[wid])
    pltpu.sync_copy(so_v, s_hbm.at[wid])


_sc_kernel = functools.partial(
    pl.kernel,
    out_type=(jax.ShapeDtypeStruct((_NW, 16), jnp.float32),
              jax.ShapeDtypeStruct((_NW, 16), jnp.float32)),
    mesh=plsc.VectorSubcoreMesh(core_axis_name="c", subcore_axis_name="s"),
    compiler_params=pltpu.CompilerParams(needs_layout_passes=False),
    scratch_types=[
        pltpu.VMEM((_QW, _D), jnp.float32),
        pltpu.VMEM((_QW, _NCH, _CH), jnp.int32),
        pltpu.VMEM((4, _CH, _D), jnp.float32),
        pltpu.VMEM((_K,), jnp.float32),
        pltpu.VMEM((16,), jnp.float32),
        pltpu.VMEM((16,), jnp.float32),
        pltpu.VMEM((16,), jnp.float32),
        pltpu.SemaphoreType.DMA((8,)),
    ],
)(_sc_body)


def kernel(idx, fea_q, fea_k, gpu_idx, bank, noise_idx):
    del idx, gpu_idx
    fkn, dl = pl.pallas_call(
        _prologue_body,
        out_shape=(
            jax.ShapeDtypeStruct((_B, _D), jnp.float32),
            jax.ShapeDtypeStruct((_B, 1), jnp.float32),
        ),
    )(fea_q, fea_k)

    nidx3 = noise_idx.reshape(_NW, _QW, _NCH, _CH)
    m, s = _sc_kernel(fea_q, nidx3, bank)
    m = m[:, :_QW].reshape(_B, 1)
    s = s[:, :_QW].reshape(_B, 1)

    loss = pl.pallas_call(
        _epilogue_body,
        out_shape=jax.ShapeDtypeStruct((1, 1), jnp.float32),
    )(m, s, dl)
    return (loss[0, 0], fkn)
